# baseline (speedup 1.0000x reference)
import functools

import jax
import jax.numpy as jnp
from jax import lax
from jax.experimental import pallas as pl
from jax.experimental.pallas import tpu as pltpu
from jax.experimental.pallas import tpu_sc as plsc

_NC = 2
_NS = 16
_NW = _NC * _NS
_CH = 128


def _cast_bf16(W, BH):
    H, K = W.shape

    def body(w_ref, o_ref):
        o_ref[...] = w_ref[...].astype(jnp.bfloat16)

    return pl.pallas_call(
        body,
        grid=(H // BH,),
        in_specs=[pl.BlockSpec((BH, K), lambda h: (h, 0))],
        out_specs=pl.BlockSpec((BH, K), lambda h: (h, 0)),
        out_shape=jax.ShapeDtypeStruct((H, K), jnp.bfloat16),
    )(W)


def _sc_gather(table, idx3):
    V, D = table.shape
    NW, NCH, CH = idx3.shape
    assert NCH % 2 == 0
    R = NW * NCH * CH
    b_per_w = NCH * CH

    mesh = plsc.VectorSubcoreMesh(core_axis_name="c", subcore_axis_name="s")

    @functools.partial(
        pl.kernel,
        mesh=mesh,
        compiler_params=pltpu.CompilerParams(use_tc_tiling_on_sc=True),
        out_type=jax.ShapeDtypeStruct((R, D), table.dtype),
        scratch_types=[
            pltpu.VMEM((NCH, CH), jnp.int32),
            pltpu.VMEM((2, CH, D), table.dtype),
            pltpu.SemaphoreType.DMA,
            pltpu.SemaphoreType.DMA,
        ],
    )
    def k(table_hbm, idx_hbm, out_hbm, idx_v, rows_v, sem0, sem1):
        c = lax.axis_index("c")
        s = lax.axis_index("s")
        wid = s * _NC + c
        base = wid * b_per_w
        pltpu.sync_copy(idx_hbm.at[wid], idx_v)
        sems = (sem0, sem1)
        pltpu.async_copy(table_hbm.at[idx_v.at[0]], rows_v.at[0], sem0)
        pltpu.async_copy(table_hbm.at[idx_v.at[1]], rows_v.at[1], sem1)

        def body(g, carry):
            for b in range(2):
                j = 2 * g + b
                pltpu.make_async_copy(
                    table_hbm.at[pl.ds(0, CH)], rows_v.at[b], sems[b]
                ).wait()
                pltpu.sync_copy(rows_v.at[b], out_hbm.at[pl.ds(base + j * CH, CH)])

                @pl.when(j + 2 < NCH)
                def _():
                    pltpu.async_copy(
                        table_hbm.at[idx_v.at[j + 2]], rows_v.at[b], sems[b]
                    )

            return carry

        lax.fori_loop(0, NCH // 2, body, 0)

    return k(table, idx3)


def _tc_mlp(rows, S, W1b, b1r, W2b, b2r):
    R, D = rows.shape
    B = R // S
    K = S * D
    HID = W1b.shape[0]
    NOUT = W2b.shape[0]
    BM = 256

    def body(x_ref, w1_ref, b1_ref, w2_ref, b2_ref, o_ref):
        x = x_ref[...].reshape(BM, K).astype(jnp.bfloat16)
        h = lax.dot_general(
            x, w1_ref[...], (((1,), (1,)), ((), ())),
            preferred_element_type=jnp.float32,
        )
        h = jnp.maximum(h + b1_ref[...], 0.0).astype(jnp.bfloat16)
        o = lax.dot_general(
            h, w2_ref[...], (((1,), (1,)), ((), ())),
            preferred_element_type=jnp.float32,
        )
        o_ref[...] = o + b2_ref[...]

    return pl.pallas_call(
        body,
        grid=(B // BM,),
        in_specs=[
            pl.BlockSpec((BM * S, D), lambda m: (m, 0)),
            pl.BlockSpec((HID, K), lambda m: (0, 0)),
            pl.BlockSpec((1, HID), lambda m: (0, 0)),
            pl.BlockSpec((NOUT, HID), lambda m: (0, 0)),
            pl.BlockSpec((1, NOUT), lambda m: (0, 0)),
        ],
        out_specs=pl.BlockSpec((BM, NOUT), lambda m: (m, 0)),
        out_shape=jax.ShapeDtypeStruct((B, NOUT), jnp.float32),
    )(rows, W1b, b1r, W2b, b2r)


def kernel(sentence, emb_table, W1, b1, W2, b2):
    B, S = sentence.shape
    V, D = emb_table.shape
    HID = W1.shape[0]
    T = W2.shape[0]
    R = B * S
    NCH = R // (_NW * _CH)

    W1b = _cast_bf16(W1, 256)
    W2b = W2.astype(jnp.bfloat16)
    b1r = b1.reshape(1, HID)
    b2r = b2.reshape(1, T)

    NCHUNK = 4
    Bc = B // NCHUNK
    CH = 80
    NCH = (Bc * S) // (_NW * CH)
    outs = []
    for c in range(NCHUNK):
        idx3 = sentence[c * Bc:(c + 1) * Bc].reshape(_NW, NCH, CH).astype(jnp.int32)
        rows = _sc_gather(emb_table, idx3)
        outs.append(_tc_mlp(rows, S, W1b, b1r, W2b, b2r))
    return jnp.concatenate(outs, axis=0)


# speedup vs baseline: 1.0923x; 1.0923x over previous
import functools

import jax
import jax.numpy as jnp
from jax import lax
from jax.experimental import pallas as pl
from jax.experimental.pallas import tpu as pltpu
from jax.experimental.pallas import tpu_sc as plsc

_NC = 2
_NS = 16
_NW = _NC * _NS
_CH = 128


def _cast_bf16(W, BH):
    H, K = W.shape

    def body(w_ref, o_ref):
        o_ref[...] = w_ref[...].astype(jnp.bfloat16)

    return pl.pallas_call(
        body,
        grid=(H // BH,),
        in_specs=[pl.BlockSpec((BH, K), lambda h: (h, 0))],
        out_specs=pl.BlockSpec((BH, K), lambda h: (h, 0)),
        out_shape=jax.ShapeDtypeStruct((H, K), jnp.bfloat16),
    )(W)


def _sc_gather(table, idx3):
    V, D = table.shape
    NW, NCH, CH = idx3.shape
    assert NCH % 2 == 0
    R = NW * NCH * CH
    b_per_w = NCH * CH

    mesh = plsc.VectorSubcoreMesh(core_axis_name="c", subcore_axis_name="s")

    @functools.partial(
        pl.kernel,
        mesh=mesh,
        compiler_params=pltpu.CompilerParams(use_tc_tiling_on_sc=True),
        out_type=jax.ShapeDtypeStruct((R, D), table.dtype),
        scratch_types=[
            pltpu.VMEM((NCH, CH), jnp.int32),
            pltpu.VMEM((2, CH, D), table.dtype),
            pltpu.SemaphoreType.DMA,
            pltpu.SemaphoreType.DMA,
        ],
    )
    def k(table_hbm, idx_hbm, out_hbm, idx_v, rows_v, sem0, sem1):
        c = lax.axis_index("c")
        s = lax.axis_index("s")
        wid = s * _NC + c
        base = wid * b_per_w
        pltpu.sync_copy(idx_hbm.at[wid], idx_v)
        sems = (sem0, sem1)
        pltpu.async_copy(table_hbm.at[idx_v.at[0]], rows_v.at[0], sem0)
        pltpu.async_copy(table_hbm.at[idx_v.at[1]], rows_v.at[1], sem1)

        def body(g, carry):
            for b in range(2):
                j = 2 * g + b
                pltpu.make_async_copy(
                    table_hbm.at[pl.ds(0, CH)], rows_v.at[b], sems[b]
                ).wait()
                pltpu.sync_copy(rows_v.at[b], out_hbm.at[pl.ds(base + j * CH, CH)])

                @pl.when(j + 2 < NCH)
                def _():
                    pltpu.async_copy(
                        table_hbm.at[idx_v.at[j + 2]], rows_v.at[b], sems[b]
                    )

            return carry

        lax.fori_loop(0, NCH // 2, body, 0)

    return k(table, idx3)


def _tc_mlp(rows, S, W1b, b1r, W2b, b2r):
    R, D = rows.shape
    B = R // S
    K = S * D
    HID = W1b.shape[0]
    NOUT = W2b.shape[0]
    BM = 256

    def body(x_ref, w1_ref, b1_ref, w2_ref, b2_ref, o_ref):
        x = x_ref[...].reshape(BM, K).astype(jnp.bfloat16)
        h = lax.dot_general(
            x, w1_ref[...], (((1,), (1,)), ((), ())),
            preferred_element_type=jnp.float32,
        )
        h = jnp.maximum(h + b1_ref[...], 0.0).astype(jnp.bfloat16)
        o = lax.dot_general(
            h, w2_ref[...], (((1,), (1,)), ((), ())),
            preferred_element_type=jnp.float32,
        )
        o_ref[...] = o + b2_ref[...]

    return pl.pallas_call(
        body,
        grid=(B // BM,),
        in_specs=[
            pl.BlockSpec((BM * S, D), lambda m: (m, 0)),
            pl.BlockSpec((HID, K), lambda m: (0, 0)),
            pl.BlockSpec((1, HID), lambda m: (0, 0)),
            pl.BlockSpec((NOUT, HID), lambda m: (0, 0)),
            pl.BlockSpec((1, NOUT), lambda m: (0, 0)),
        ],
        out_specs=pl.BlockSpec((BM, NOUT), lambda m: (m, 0)),
        out_shape=jax.ShapeDtypeStruct((B, NOUT), jnp.float32),
    )(rows, W1b, b1r, W2b, b2r)


def kernel(sentence, emb_table, W1, b1, W2, b2):
    B, S = sentence.shape
    V, D = emb_table.shape
    HID = W1.shape[0]
    T = W2.shape[0]
    R = B * S
    NCH = R // (_NW * _CH)

    W1b = _cast_bf16(W1, 128)
    W2b = _cast_bf16(W2, T)
    b1r = b1.reshape(1, HID)
    b2r = b2.reshape(1, T)

    NCHUNK = 2
    Bc = B // NCHUNK
    CH = 64
    NCH = (Bc * S) // (_NW * CH)
    outs = []
    for c in range(NCHUNK):
        idx3 = sentence[c * Bc:(c + 1) * Bc].reshape(_NW, NCH, CH).astype(jnp.int32)
        rows = _sc_gather(emb_table, idx3)
        outs.append(_tc_mlp(rows, S, W1b, b1r, W2b, b2r))
    return jnp.concatenate(outs, axis=0)


# speedup vs baseline: 1.1180x; 1.0235x over previous
import functools

import jax
import jax.numpy as jnp
from jax import lax
from jax.experimental import pallas as pl
from jax.experimental.pallas import tpu as pltpu
from jax.experimental.pallas import tpu_sc as plsc

_NC = 2
_NS = 16
_NW = _NC * _NS


def _cast_bf16(W, BH):
    H, K = W.shape

    def body(w_ref, o_ref):
        o_ref[...] = w_ref[...].astype(jnp.bfloat16)

    return pl.pallas_call(
        body,
        grid=(H // BH,),
        in_specs=[pl.BlockSpec((BH, K), lambda h: (h, 0))],
        out_specs=pl.BlockSpec((BH, K), lambda h: (h, 0)),
        out_shape=jax.ShapeDtypeStruct((H, K), jnp.bfloat16),
    )(W)


def _sc_gather_tiled(table, idx3, B, S):
    V, D = table.shape
    NW, NCH, CH = idx3.shape
    assert NCH % 2 == 0 and CH % 8 == 0
    b_per_w = NCH * CH
    NT = CH // 8

    mesh = plsc.VectorSubcoreMesh(core_axis_name="c", subcore_axis_name="s")

    @functools.partial(
        pl.kernel,
        mesh=mesh,
        compiler_params=pltpu.CompilerParams(use_tc_tiling_on_sc=True),
        out_type=jax.ShapeDtypeStruct((B, S * D), table.dtype),
        scratch_types=[
            pltpu.VMEM((NCH, CH), jnp.int32),
            pltpu.VMEM((2, CH, D), table.dtype),
            pltpu.SemaphoreType.DMA,
            pltpu.SemaphoreType.DMA,
            pltpu.SemaphoreType.DMA,
            pltpu.SemaphoreType.DMA,
        ],
    )
    def k(table_hbm, idx_hbm, out_hbm, idx_v, rows_v, sem0, sem1, osem0, osem1):
        c = lax.axis_index("c")
        s = lax.axis_index("s")
        wid = s * _NC + c
        base = wid * b_per_w
        pltpu.sync_copy(idx_hbm.at[wid], idx_v)
        sems = (sem0, sem1)
        osems = (osem0, osem1)
        pltpu.async_copy(table_hbm.at[idx_v.at[0]], rows_v.at[0], sem0)
        pltpu.async_copy(table_hbm.at[idx_v.at[1]], rows_v.at[1], sem1)

        def body(g, carry):
            for b in range(2):
                j = 2 * g + b
                pltpu.make_async_copy(
                    table_hbm.at[pl.ds(0, CH)], rows_v.at[b], sems[b]
                ).wait()
                t0 = (base + j * CH) // 8
                for kk in range(NT):
                    t = t0 + kk
                    bt = t // S
                    cc = t % S
                    pltpu.async_copy(
                        rows_v.at[b, pl.ds(kk * 8, 8)],
                        out_hbm.at[pl.ds(bt * 8, 8), pl.ds(cc * D, D)],
                        osems[b],
                    )
                pltpu.make_async_copy(
                    table_hbm.at[pl.ds(0, CH)], rows_v.at[b], osems[b]
                ).wait()

                @pl.when(j + 2 < NCH)
                def _():
                    pltpu.async_copy(
                        table_hbm.at[idx_v.at[j + 2]], rows_v.at[b], sems[b]
                    )

            return carry

        lax.fori_loop(0, NCH // 2, body, 0)

    return k(table, idx3)


def _tc_mlp(flat, W1b, b1r, W2b, b2r):
    B, K = flat.shape
    HID = W1b.shape[0]
    NOUT = W2b.shape[0]
    BM = 256

    def body(x_ref, w1_ref, b1_ref, w2_ref, b2_ref, o_ref):
        x = x_ref[...].astype(jnp.bfloat16)
        h = lax.dot_general(
            x, w1_ref[...], (((1,), (1,)), ((), ())),
            preferred_element_type=jnp.float32,
        )
        h = jnp.maximum(h + b1_ref[...], 0.0).astype(jnp.bfloat16)
        o = lax.dot_general(
            h, w2_ref[...], (((1,), (1,)), ((), ())),
            preferred_element_type=jnp.float32,
        )
        o_ref[...] = o + b2_ref[...]

    return pl.pallas_call(
        body,
        grid=(B // BM,),
        in_specs=[
            pl.BlockSpec((BM, K), lambda m: (m, 0)),
            pl.BlockSpec((HID, K), lambda m: (0, 0)),
            pl.BlockSpec((1, HID), lambda m: (0, 0)),
            pl.BlockSpec((NOUT, HID), lambda m: (0, 0)),
            pl.BlockSpec((1, NOUT), lambda m: (0, 0)),
        ],
        out_specs=pl.BlockSpec((BM, NOUT), lambda m: (m, 0)),
        out_shape=jax.ShapeDtypeStruct((B, NOUT), jnp.float32),
    )(flat, W1b, b1r, W2b, b2r)


def kernel(sentence, emb_table, W1, b1, W2, b2):
    B, S = sentence.shape
    V, D = emb_table.shape
    HID = W1.shape[0]
    T = W2.shape[0]

    W1b = _cast_bf16(W1, 128)
    W2b = _cast_bf16(W2, T)
    b1r = b1.reshape(1, HID)
    b2r = b2.reshape(1, T)

    NCHUNK = 2
    Bc = B // NCHUNK
    CH = 64
    NCH = (Bc * S) // (_NW * CH)
    outs = []
    for c in range(NCHUNK):
        sent_c = sentence[c * Bc:(c + 1) * Bc].astype(jnp.int32)
        idx3 = (
            sent_c.reshape(Bc // 8, 8, S)
            .transpose(0, 2, 1)
            .reshape(_NW, NCH, CH)
        )
        flat = _sc_gather_tiled(emb_table, idx3, Bc, S)
        outs.append(_tc_mlp(flat, W1b, b1r, W2b, b2r))
    return jnp.concatenate(outs, axis=0)
